# Initial kernel scaffold; baseline (speedup 1.0000x reference)
import jax
import jax.numpy as jnp
from jax.experimental import pallas as pl


def kernel(inputs, edge_index, W_ih, W_hh, b_ih, b_hh):
    raise NotImplementedError("write your pallas kernel here")


# speedup vs baseline: 3.3956x; 3.3956x over previous
import functools

import jax
import jax.numpy as jnp
from jax import lax
from jax.experimental import pallas as pl
from jax.experimental.pallas import tpu as pltpu
from jax.experimental.pallas import tpu_sc as plsc

N = 10000
E = 320000
D = 128

NC = 2
NS = 16

CHUNK = 128
E_PAD = 327680
NCHUNK = E_PAD // CHUNK
CH_PER_TILE = NCHUNK // (NC * NS)
N_PAD = 10240
ROWS_PER_TILE = N_PAD // NS


def _sc_segment_sum(inputs, src2d, dst2d, zeros):
  mesh = plsc.VectorSubcoreMesh(core_axis_name="c", subcore_axis_name="s")

  @functools.partial(
      pl.kernel,
      out_type=jax.ShapeDtypeStruct((NC, N_PAD, D), jnp.float32),
      mesh=mesh,
      scratch_types=[
          pltpu.VMEM((CH_PER_TILE, CHUNK), jnp.int32),
          pltpu.VMEM((CH_PER_TILE, CHUNK), jnp.int32),
          pltpu.VMEM((CHUNK, D), jnp.float32),
          pltpu.VMEM_SHARED((N_PAD, D), jnp.float32),
          pltpu.SemaphoreType.DMA,
      ],
  )
  def seg_sum(inp_hbm, src_hbm, dst_hbm, zero_hbm, out_hbm,
              src_v, dst_v, rows_v, acc, sem):
    cid = lax.axis_index("c")
    sid = lax.axis_index("s")
    wid = cid * NS + sid

    pltpu.sync_copy(src_hbm.at[pl.ds(wid * CH_PER_TILE, CH_PER_TILE)], src_v)
    pltpu.sync_copy(dst_hbm.at[pl.ds(wid * CH_PER_TILE, CH_PER_TILE)], dst_v)

    pltpu.sync_copy(zero_hbm.at[pl.ds(sid * ROWS_PER_TILE, ROWS_PER_TILE)],
                    acc.at[pl.ds(sid * ROWS_PER_TILE, ROWS_PER_TILE)])
    plsc.subcore_barrier()

    def body(j, carry):
      pltpu.async_copy(inp_hbm.at[src_v.at[j]], rows_v, sem).wait()
      pltpu.sync_copy(rows_v, acc.at[dst_v.at[j]], add=True)
      return carry

    lax.fori_loop(0, CH_PER_TILE, body, 0, unroll=False)
    plsc.subcore_barrier()

    pltpu.sync_copy(acc.at[pl.ds(sid * ROWS_PER_TILE, ROWS_PER_TILE)],
                    out_hbm.at[cid, pl.ds(sid * ROWS_PER_TILE, ROWS_PER_TILE)])

  return seg_sum(inputs, src2d, dst2d, zeros)


BN = 400


def _lstm_block(p_ref, wt_ref, b_ref, o_ref):
  h = p_ref[0] + p_ref[1]
  gates = jnp.dot(h, wt_ref[...], preferred_element_type=jnp.float32)
  gates = gates + b_ref[...]
  i_g = jax.nn.sigmoid(gates[:, 0 * D:1 * D])
  g_g = jnp.tanh(gates[:, 2 * D:3 * D])
  o_g = jax.nn.sigmoid(gates[:, 3 * D:4 * D])
  c_new = i_g * g_g
  o_ref[...] = jnp.maximum(o_g * jnp.tanh(c_new), 0.0)


def _tc_lstm(partials, wt, bsum):
  return pl.pallas_call(
      _lstm_block,
      grid=(N // BN,),
      in_specs=[
          pl.BlockSpec((2, BN, D), lambda i: (0, i, 0)),
          pl.BlockSpec((D, 4 * D), lambda i: (0, 0)),
          pl.BlockSpec((1, 4 * D), lambda i: (0, 0)),
      ],
      out_specs=pl.BlockSpec((BN, D), lambda i: (i, 0)),
      out_shape=jax.ShapeDtypeStruct((N, D), jnp.float32),
  )(partials, wt, bsum)


def kernel(inputs, edge_index, W_ih, W_hh, b_ih, b_hh):
  src = edge_index[0].astype(jnp.int32)
  dst = edge_index[1].astype(jnp.int32)
  src_p = jnp.concatenate([src, jnp.zeros((E_PAD - E,), jnp.int32)])
  dst_p = jnp.concatenate([dst, jnp.full((E_PAD - E,), N, jnp.int32)])
  src2d = src_p.reshape(NCHUNK, CHUNK)
  dst2d = dst_p.reshape(NCHUNK, CHUNK)
  zeros = jnp.zeros((N_PAD, D), jnp.float32)

  partials = _sc_segment_sum(inputs, src2d, dst2d, zeros)

  wt = W_ih.T
  bsum = (b_ih + b_hh).reshape(1, 4 * D)
  return _tc_lstm(partials, wt, bsum)


# speedup vs baseline: 3.5915x; 1.0577x over previous
import functools

import jax
import jax.numpy as jnp
from jax import lax
from jax.experimental import pallas as pl
from jax.experimental.pallas import tpu as pltpu
from jax.experimental.pallas import tpu_sc as plsc

N = 10000
E = 320000
D = 128

NC = 2
NS = 16

CHUNK = 64
CH_PER_TILE = 160
NPASS = 2
CH_PASS = CH_PER_TILE // NPASS
E_PAD = NC * NS * CH_PER_TILE * CHUNK
N_PAD = 10112
ROWS_PER_TILE = N_PAD // NS


def _sc_segment_sum(inputs, src2d, dst2d, zeros):
  mesh = plsc.VectorSubcoreMesh(core_axis_name="c", subcore_axis_name="s")

  @functools.partial(
      pl.kernel,
      out_type=jax.ShapeDtypeStruct((NC, N_PAD, D), jnp.float32),
      mesh=mesh,
      scratch_types=[
          pltpu.VMEM((2, CH_PASS, CHUNK), jnp.int32),
          pltpu.VMEM((2, CHUNK, D), jnp.float32),
          pltpu.VMEM_SHARED((N_PAD, D), jnp.float32),
          pltpu.SemaphoreType.DMA,
          pltpu.SemaphoreType.DMA,
          pltpu.SemaphoreType.DMA,
          pltpu.SemaphoreType.DMA,
      ],
  )
  def seg_sum(inp_hbm, src_hbm, dst_hbm, zero_hbm, out_hbm,
              idx_v, rows_v, acc, gs0, gs1, ss0, ss1):
    cid = lax.axis_index("c")
    sid = lax.axis_index("s")
    wid = cid * NS + sid

    pltpu.sync_copy(zero_hbm.at[pl.ds(sid * ROWS_PER_TILE, ROWS_PER_TILE)],
                    acc.at[pl.ds(sid * ROWS_PER_TILE, ROWS_PER_TILE)])
    plsc.subcore_barrier()

    rb0, rb1 = rows_v.at[0], rows_v.at[1]

    def wait_gather(rb, gs):
      pltpu.make_async_copy(inp_hbm.at[pl.ds(0, CHUNK)], rb, gs).wait()

    def wait_scatter(rb, ss):
      pltpu.make_async_copy(rb, acc.at[pl.ds(0, CHUNK)], ss).wait()

    npair = CH_PASS // 2

    def one_pass(p, carry):
      pltpu.sync_copy(src_hbm.at[wid, pl.ds(p * CH_PASS, CH_PASS)],
                      idx_v.at[0])
      pltpu.sync_copy(dst_hbm.at[wid, pl.ds(p * CH_PASS, CH_PASS)],
                      idx_v.at[1])
      pltpu.async_copy(inp_hbm.at[idx_v.at[0, 0]], rb0, gs0)
      pltpu.async_copy(inp_hbm.at[idx_v.at[0, 1]], rb1, gs1)

      def body(jj, carry2):
        j = 2 * jj
        wait_gather(rb0, gs0)
        pltpu.async_copy(rb0, acc.at[idx_v.at[1, j]], ss0, add=True)
        wait_gather(rb1, gs1)
        pltpu.async_copy(rb1, acc.at[idx_v.at[1, j + 1]], ss1, add=True)
        wait_scatter(rb0, ss0)

        @pl.when(jj < npair - 1)
        def _():
          pltpu.async_copy(inp_hbm.at[idx_v.at[0, j + 2]], rb0, gs0)

        wait_scatter(rb1, ss1)

        @pl.when(jj < npair - 1)
        def _():
          pltpu.async_copy(inp_hbm.at[idx_v.at[0, j + 3]], rb1, gs1)

        return carry2

      lax.fori_loop(0, npair, body, 0, unroll=False)
      return carry

    lax.fori_loop(0, NPASS, one_pass, 0, unroll=False)
    plsc.subcore_barrier()

    pltpu.sync_copy(acc.at[pl.ds(sid * ROWS_PER_TILE, ROWS_PER_TILE)],
                    out_hbm.at[cid, pl.ds(sid * ROWS_PER_TILE, ROWS_PER_TILE)])

  return seg_sum(inputs, src2d, dst2d, zeros)


BN = 400


def _lstm_block(p_ref, wt_ref, b_ref, o_ref):
  h = p_ref[0] + p_ref[1]
  gates = jnp.dot(h, wt_ref[...], preferred_element_type=jnp.float32)
  gates = gates + b_ref[...]
  i_g = jax.nn.sigmoid(gates[:, 0 * D:1 * D])
  g_g = jnp.tanh(gates[:, 2 * D:3 * D])
  o_g = jax.nn.sigmoid(gates[:, 3 * D:4 * D])
  c_new = i_g * g_g
  o_ref[...] = jnp.maximum(o_g * jnp.tanh(c_new), 0.0)


def _tc_lstm(partials, wt, bsum):
  return pl.pallas_call(
      _lstm_block,
      grid=(N // BN,),
      in_specs=[
          pl.BlockSpec((2, BN, D), lambda i: (0, i, 0)),
          pl.BlockSpec((D, 4 * D), lambda i: (0, 0)),
          pl.BlockSpec((1, 4 * D), lambda i: (0, 0)),
      ],
      out_specs=pl.BlockSpec((BN, D), lambda i: (i, 0)),
      out_shape=jax.ShapeDtypeStruct((N, D), jnp.float32),
  )(partials, wt, bsum)


def kernel(inputs, edge_index, W_ih, W_hh, b_ih, b_hh):
  src = edge_index[0].astype(jnp.int32)
  dst = edge_index[1].astype(jnp.int32)
  nw = NC * NS
  src3d = jnp.concatenate(
      [src, jnp.zeros((E_PAD - E,), jnp.int32)]).reshape(nw, CH_PER_TILE, CHUNK)
  dst3d = jnp.concatenate(
      [dst, jnp.full((E_PAD - E,), N, jnp.int32)]).reshape(nw, CH_PER_TILE, CHUNK)
  zeros = jnp.zeros((N_PAD, D), jnp.float32)

  partials = _sc_segment_sum(inputs, src3d, dst3d, zeros)

  wt = W_ih.T
  bsum = (b_ih + b_hh).reshape(1, 4 * D)
  return _tc_lstm(partials, wt, bsum)
